# baseline (speedup 1.0000x reference)
import functools

import jax
import jax.numpy as jnp
from jax import lax
from jax.experimental import pallas as pl
from jax.experimental.pallas import tpu as pltpu
from jax.experimental.pallas import tpu_sc as plsc

VOCAB = 100000
DIM = 128
L = 50
NEW_ROWS = 513
NEW_PAD = 513
B = 4096
N = B * L
NC, NS = 2, 16
NW = NC * NS
PER_W = N // NW
EX_W = PER_W // L
EPC = 1
CHUNK = EPC * L
NCHUNK = PER_W // CHUNK
GROUPS = (CHUNK + 15) // 16
NBUF = 8

_mesh = plsc.VectorSubcoreMesh(core_axis_name="c", subcore_axis_name="s",
                               num_cores=NC, num_subcores=NS)


@functools.partial(
    pl.kernel,
    out_type=jax.ShapeDtypeStruct((B, L, DIM), jnp.float32),
    mesh=_mesh,
    compiler_params=pltpu.CompilerParams(needs_layout_passes=False),
    scratch_types=[
        pltpu.VMEM((PER_W + 16,), jnp.int32),
        pltpu.VMEM((NEW_PAD * DIM,), jnp.float32),
        [pltpu.VMEM((GROUPS * 16,), jnp.int32)] * NBUF,
        [pltpu.VMEM((CHUNK, DIM), jnp.float32)] * NBUF,
        [pltpu.SemaphoreType.DMA] * NBUF,
        [pltpu.SemaphoreType.DMA] * NBUF,
    ],
)
def _emb_lookup(x_hbm, orig_hbm, new_hbm, out_hbm,
                idx_v, newtab_v, idx1_v, rows_v, gsem, wsem):
    wid = lax.axis_index("s") * NC + lax.axis_index("c")
    base_w = wid * PER_W
    ex_w = wid * EX_W
    pltpu.sync_copy(x_hbm.at[pl.ds(base_w, PER_W)], idx_v.at[pl.ds(0, PER_W)])
    idx_v[pl.ds(PER_W, 16)] = jnp.zeros((16,), jnp.int32)
    pltpu.sync_copy(new_hbm, newtab_v.at[pl.ds(0, NEW_ROWS * DIM)])
    lane = lax.iota(jnp.int32, 16)

    def fire(c, b):
        for g in range(GROUPS):
            v = idx_v[pl.ds(c * CHUNK + g * 16, 16)]
            idx1_v[b][pl.ds(g * 16, 16)] = jnp.minimum(v, VOCAB - 1)
        pltpu.async_copy(orig_hbm.at[idx1_v[b].at[pl.ds(0, CHUNK)]],
                         rows_v[b], gsem[b])

    def fixup(c, b):
        acc = idx_v[pl.ds(c * CHUNK, 16)]
        for g in range(1, GROUPS):
            acc = jnp.maximum(acc, idx_v[pl.ds(c * CHUNK + g * 16, 16)])
        chunk_dirty = plsc.all_reduce_population_count(acc >= VOCAB)[0]

        @pl.when(chunk_dirty > 0)
        def _chunk():
            def gbody(g, _):
                gbase = c * CHUNK + g * 16
                v = idx_v[pl.ds(gbase, 16)]
                nlanes = jnp.where(g < GROUPS - 1, 16, CHUNK - 16 * (GROUPS - 1))
                m = (v >= VOCAB) & (lane < nlanes)
                n_off = plsc.all_reduce_population_count(m)[0]

                @pl.when(n_off > 0)
                def _fix():
                    nbase = jnp.maximum(v - (VOCAB - 1), 0) * DIM
                    rows16 = g * 16 + lane

                    def cbody(jc, _):
                        for u in range(8):
                            col = jc * 8 + u
                            colv = jnp.full((16,), 0, jnp.int32) + col
                            vals = plsc.load_gather(newtab_v, [nbase + col])
                            plsc.store_scatter(rows_v[b], [rows16, colv],
                                               vals, mask=m)
                        return ()

                    lax.fori_loop(0, DIM // 8, cbody, ())
                return ()

            lax.fori_loop(0, GROUPS, gbody, ())

    def write(c, b):
        for e in range(EPC):
            pltpu.async_copy(rows_v[b].at[pl.ds(e * L, L)],
                             out_hbm.at[ex_w + c * EPC + e], wsem[b])

    def wait_gather(b):
        pltpu.make_async_copy(orig_hbm.at[idx1_v[b].at[pl.ds(0, CHUNK)]],
                              rows_v[b], gsem[b]).wait()

    def wait_write(b):
        for e in range(EPC):
            pltpu.make_async_copy(rows_v[b].at[pl.ds(e * L, L)],
                                  out_hbm.at[ex_w], wsem[b]).wait()

    for p in range(NBUF - 1):
        fire(p, p)

    def block_body(blk, _):
        for j in range(NBUF):
            c = blk * NBUF + j
            wait_gather(j)
            fixup(c, j)
            write(c, j)
            nb = (j + NBUF - 1) % NBUF

            @pl.when(c >= 1)
            def _drain(nb=nb):
                wait_write(nb)

            @pl.when(c + NBUF - 1 < NCHUNK)
            def _pref(c=c, nb=nb):
                fire(c + NBUF - 1, nb)
        return ()

    lax.fori_loop(0, NCHUNK // NBUF, block_body, ())
    wait_write((NCHUNK - 1) % NBUF)


def kernel(x, orig_weight, new_weight):
    return _emb_lookup(x.reshape(-1), orig_weight, new_weight.reshape(-1))


# speedup vs baseline: 1.0041x; 1.0041x over previous
import functools

import jax
import jax.numpy as jnp
from jax import lax
from jax.experimental import pallas as pl
from jax.experimental.pallas import tpu as pltpu
from jax.experimental.pallas import tpu_sc as plsc

VOCAB = 100000
DIM = 128
L = 50
NEW_ROWS = 513
NEW_PAD = 520
B = 4096
N = B * L
NC, NS = 2, 16
NW = NC * NS
PER_W = N // NW
EX_W = PER_W // L
EPC = 2
CHUNK = EPC * L
NCHUNK = PER_W // CHUNK
GROUPS = (CHUNK + 15) // 16
NBUF = 4

_mesh = plsc.VectorSubcoreMesh(core_axis_name="c", subcore_axis_name="s",
                               num_cores=NC, num_subcores=NS)


@functools.partial(
    pl.kernel,
    out_type=jax.ShapeDtypeStruct((B, L, DIM), jnp.float32),
    mesh=_mesh,
    compiler_params=pltpu.CompilerParams(needs_layout_passes=False),
    scratch_types=[
        pltpu.VMEM((PER_W + 16,), jnp.int32),
        pltpu.VMEM((NEW_PAD * DIM,), jnp.float32),
        [pltpu.VMEM((GROUPS * 16,), jnp.int32)] * NBUF,
        [pltpu.VMEM((CHUNK, DIM), jnp.float32)] * NBUF,
        [pltpu.SemaphoreType.DMA] * NBUF,
        [pltpu.SemaphoreType.DMA] * NBUF,
    ],
)
def _emb_lookup(x_hbm, orig_hbm, new_hbm, out_hbm,
                idx_v, newtab_v, idx1_v, rows_v, gsem, wsem):
    wid = lax.axis_index("s") * NC + lax.axis_index("c")
    base_w = wid * PER_W
    ex_w = wid * EX_W
    pltpu.sync_copy(x_hbm.at[pl.ds(base_w, PER_W)], idx_v.at[pl.ds(0, PER_W)])
    idx_v[pl.ds(PER_W, 16)] = jnp.zeros((16,), jnp.int32)
    pltpu.sync_copy(new_hbm, newtab_v.at[pl.ds(0, NEW_ROWS * DIM)])
    lane = lax.iota(jnp.int32, 16)

    def fire(c, b):
        for g in range(GROUPS):
            v = idx_v[pl.ds(c * CHUNK + g * 16, 16)]
            idx1_v[b][pl.ds(g * 16, 16)] = jnp.minimum(v, VOCAB - 1)
        pltpu.async_copy(orig_hbm.at[idx1_v[b].at[pl.ds(0, CHUNK)]],
                         rows_v[b], gsem[b])

    def fixup(c, b):
        acc = idx_v[pl.ds(c * CHUNK, 16)]
        for g in range(1, GROUPS):
            acc = jnp.maximum(acc, idx_v[pl.ds(c * CHUNK + g * 16, 16)])
        chunk_dirty = plsc.all_reduce_population_count(acc >= VOCAB)[0]

        @pl.when(chunk_dirty > 0)
        def _chunk():
            def gbody(g, _):
                gbase = c * CHUNK + g * 16
                v = idx_v[pl.ds(gbase, 16)]
                nlanes = jnp.where(g < GROUPS - 1, 16, CHUNK - 16 * (GROUPS - 1))
                m = (v >= VOCAB) & (lane < nlanes)
                n_off = plsc.all_reduce_population_count(m)[0]

                @pl.when(n_off > 0)
                def _fix():
                    nbase = jnp.maximum(v - (VOCAB - 1), 0) * DIM
                    rows16 = g * 16 + lane

                    def cbody(jc, _):
                        for u in range(8):
                            col = jc * 8 + u
                            colv = jnp.full((16,), 0, jnp.int32) + col
                            vals = plsc.load_gather(newtab_v, [nbase + col])
                            plsc.store_scatter(rows_v[b], [rows16, colv],
                                               vals, mask=m)
                        return ()

                    lax.fori_loop(0, DIM // 8, cbody, ())
                return ()

            lax.fori_loop(0, GROUPS, gbody, ())

    def write(c, b):
        for e in range(EPC):
            pltpu.async_copy(rows_v[b].at[pl.ds(e * L, L)],
                             out_hbm.at[ex_w + c * EPC + e], wsem[b])

    def wait_gather(b):
        pltpu.make_async_copy(orig_hbm.at[idx1_v[b].at[pl.ds(0, CHUNK)]],
                              rows_v[b], gsem[b]).wait()

    def wait_write(b):
        for e in range(EPC):
            pltpu.make_async_copy(rows_v[b].at[pl.ds(e * L, L)],
                                  out_hbm.at[ex_w], wsem[b]).wait()

    fire(0, 0)
    fire(1, 1)
    fire(2, 2)

    def block_body(blk, _):
        for j in range(NBUF):
            c = blk * NBUF + j
            wait_gather(j)
            fixup(c, j)
            write(c, j)
            nb = (j + 3) % NBUF

            @pl.when(c >= 1)
            def _drain(nb=nb):
                wait_write(nb)

            @pl.when(c + 3 < NCHUNK)
            def _pref(c=c, nb=nb):
                fire(c + 3, nb)
        return ()

    lax.fori_loop(0, NCHUNK // NBUF, block_body, ())
    wait_write((NCHUNK - 1) % NBUF)


def kernel(x, orig_weight, new_weight):
    return _emb_lookup(x.reshape(-1), orig_weight, new_weight.reshape(-1))


# speedup vs baseline: 1.0080x; 1.0039x over previous
import functools

import jax
import jax.numpy as jnp
from jax import lax
from jax.experimental import pallas as pl
from jax.experimental.pallas import tpu as pltpu
from jax.experimental.pallas import tpu_sc as plsc

VOCAB = 100000
DIM = 128
L = 50
NEW_ROWS = 513
NEW_PAD = 520
B = 4096
N = B * L
NC, NS = 2, 16
NW = NC * NS
PER_W = N // NW
EX_W = PER_W // L
EPC = 2
CHUNK = EPC * L
NCHUNK = PER_W // CHUNK
GROUPS = (CHUNK + 15) // 16
NBUF = 4

@functools.cache
def _build():
    mesh = plsc.VectorSubcoreMesh(core_axis_name="c", subcore_axis_name="s",
                                  num_cores=NC, num_subcores=NS)
    return functools.partial(
        pl.kernel,
        out_type=jax.ShapeDtypeStruct((B, L, DIM), jnp.float32),
        mesh=mesh,
        compiler_params=pltpu.CompilerParams(needs_layout_passes=False),
        scratch_types=[
            pltpu.VMEM((PER_W + 16,), jnp.int32),
            pltpu.VMEM((NEW_PAD * DIM,), jnp.float32),
            [pltpu.VMEM((GROUPS * 16,), jnp.int32)] * NBUF,
            [pltpu.VMEM((CHUNK, DIM), jnp.float32)] * NBUF,
            [pltpu.SemaphoreType.DMA] * NBUF,
            [pltpu.SemaphoreType.DMA] * NBUF,
        ],
    )(_emb_lookup)


def _emb_lookup(x_hbm, orig_hbm, new_hbm, out_hbm,
                idx_v, newtab_v, idx1_v, rows_v, gsem, wsem):
    wid = lax.axis_index("s") * NC + lax.axis_index("c")
    base_w = wid * PER_W
    ex_w = wid * EX_W
    pltpu.sync_copy(x_hbm.at[pl.ds(base_w, PER_W)], idx_v.at[pl.ds(0, PER_W)])
    idx_v[pl.ds(PER_W, 16)] = jnp.zeros((16,), jnp.int32)
    pltpu.sync_copy(new_hbm, newtab_v.at[pl.ds(0, NEW_ROWS * DIM)])
    lane = lax.iota(jnp.int32, 16)

    def fire(c, b):
        for g in range(GROUPS):
            v = idx_v[pl.ds(c * CHUNK + g * 16, 16)]
            idx1_v[b][pl.ds(g * 16, 16)] = jnp.minimum(v, VOCAB - 1)
        pltpu.async_copy(orig_hbm.at[idx1_v[b].at[pl.ds(0, CHUNK)]],
                         rows_v[b], gsem[b])

    def fixup(c, b):
        acc = idx_v[pl.ds(c * CHUNK, 16)]
        for g in range(1, GROUPS):
            acc = jnp.maximum(acc, idx_v[pl.ds(c * CHUNK + g * 16, 16)])
        chunk_dirty = plsc.all_reduce_population_count(acc >= VOCAB)[0]

        @pl.when(chunk_dirty > 0)
        def _chunk():
            def gbody(g, _):
                gbase = c * CHUNK + g * 16
                v = idx_v[pl.ds(gbase, 16)]
                nlanes = jnp.where(g < GROUPS - 1, 16, CHUNK - 16 * (GROUPS - 1))
                m = (v >= VOCAB) & (lane < nlanes)
                n_off = plsc.all_reduce_population_count(m)[0]

                @pl.when(n_off > 0)
                def _fix():
                    nbase = jnp.maximum(v - (VOCAB - 1), 0) * DIM
                    rows16 = g * 16 + lane

                    def cbody(jc, _):
                        for u in range(8):
                            col = jc * 8 + u
                            colv = jnp.full((16,), 0, jnp.int32) + col
                            vals = plsc.load_gather(newtab_v, [nbase + col])
                            plsc.store_scatter(rows_v[b], [rows16, colv],
                                               vals, mask=m)
                        return ()

                    lax.fori_loop(0, DIM // 8, cbody, ())
                return ()

            lax.fori_loop(0, GROUPS, gbody, ())

    def write(c, b):
        for e in range(EPC):
            pltpu.async_copy(rows_v[b].at[pl.ds(e * L, L)],
                             out_hbm.at[ex_w + c * EPC + e], wsem[b])

    def wait_gather(b):
        pltpu.make_async_copy(orig_hbm.at[idx1_v[b].at[pl.ds(0, CHUNK)]],
                              rows_v[b], gsem[b]).wait()

    def wait_write(b):
        for e in range(EPC):
            pltpu.make_async_copy(rows_v[b].at[pl.ds(e * L, L)],
                                  out_hbm.at[ex_w], wsem[b]).wait()

    fire(0, 0)
    fire(1, 1)
    fire(2, 2)

    def block_body(blk, _):
        for j in range(NBUF):
            c = blk * NBUF + j
            wait_gather(j)
            fixup(c, j)
            write(c, j)
            nb = (j + 3) % NBUF

            @pl.when(c >= 1)
            def _drain(nb=nb):
                wait_write(nb)

            @pl.when(c + 3 < NCHUNK)
            def _pref(c=c, nb=nb):
                fire(c + 3, nb)
        return ()

    lax.fori_loop(0, NCHUNK // NBUF, block_body, ())
    wait_write((NCHUNK - 1) % NBUF)


def kernel(x, orig_weight, new_weight):
    return _build()(x.reshape(-1), orig_weight, new_weight.reshape(-1))
